# baseline (speedup 1.0000x reference)
import functools

import jax
import jax.numpy as jnp
from jax import lax
from jax.experimental import pallas as pl
from jax.experimental.pallas import tpu as pltpu
from jax.experimental.pallas import tpu_sc as plsc

N_WORKERS = 32
CHUNKS = 10
CHUNK = 640
EMB = 64


SUB = CHUNK // 128


def _emb_gather(x_hbm, table_hbm, out_hbm, idx_v, rows0, rows1, sem0, sem1):
    wid = lax.axis_index("s") * 2 + lax.axis_index("c")
    pltpu.sync_copy(x_hbm.at[wid], idx_v)

    bufs = (rows0, rows1)
    sems = (sem0, sem1)

    def fire(j, b):
        cps = []
        for k in range(SUB):
            cps.append(pltpu.async_copy(
                table_hbm.at[idx_v.at[j, k]],
                bufs[b].at[pl.ds(k * 128, 128)], sems[b]))
        return cps

    copies = [None, None]
    copies[0] = fire(0, 0)
    for j in range(CHUNKS):
        b = j % 2
        if j + 1 < CHUNKS:
            b2 = (j + 1) % 2
            copies[b2] = fire(j + 1, b2)
        for cp in copies[b]:
            cp.wait()
        pltpu.sync_copy(bufs[b], out_hbm.at[wid, j])


def kernel(x, table):
    mesh = plsc.VectorSubcoreMesh(core_axis_name="c", subcore_axis_name="s")
    run = functools.partial(
        pl.kernel,
        out_type=jax.ShapeDtypeStruct((N_WORKERS, CHUNKS, CHUNK, EMB),
                                      jnp.float32),
        mesh=mesh,
        scratch_types=[
            pltpu.VMEM((CHUNKS, SUB, 128), jnp.int32),
            pltpu.VMEM((CHUNK, EMB), jnp.float32),
            pltpu.VMEM((CHUNK, EMB), jnp.float32),
            pltpu.SemaphoreType.DMA,
            pltpu.SemaphoreType.DMA,
        ],
        compiler_params=pltpu.CompilerParams(use_tc_tiling_on_sc=False),
    )(_emb_gather)
    x3 = x.reshape(N_WORKERS, CHUNKS, SUB, 128)
    out = run(x3, table)
    return out.reshape(x.shape[0], x.shape[1], EMB)
